# baseline (speedup 1.0000x reference)
import jax
import jax.numpy as jnp
from jax.experimental import pallas as pl
from jax.experimental.pallas import tpu as pltpu
from jax.experimental.pallas import tpu_sc as plsc


def kernel(input, weight):
    B, S = input.shape
    N, D1, D2 = weight.shape
    D = D1 * D2
    num_indices = B * S

    table = weight.reshape(N, D)
    idx = input.reshape(1, num_indices).astype(jnp.int32)

    WINDOW = 256
    assert num_indices % WINDOW == 0

    vmesh = plsc.VectorSubcoreMesh(
        core_axis_name="core", subcore_axis_name="subcore"
    )

    @pl.kernel(
        out_type=jax.ShapeDtypeStruct((num_indices, D), weight.dtype),
        mesh=vmesh,
    )
    def sc_gather(x_hbm, i_hbm, o_hbm):
        def body(i_vmem, o_vmem):
            pltpu.sync_copy(x_hbm.at[i_vmem.at[0]], o_vmem)

        pltpu.emit_pipeline(
            body,
            grid=(num_indices // WINDOW,),
            in_specs=[
                pl.BlockSpec((1, WINDOW), index_map=lambda i: (0, i))
            ],
            out_specs=[
                pl.BlockSpec((WINDOW, D), index_map=lambda i: (i, 0))
            ],
            core_axis_name=("core", "subcore"),
            dimension_semantics=(pltpu.PARALLEL,),
        )(i_hbm, o_hbm)

    BB = 32

    def relayout_body(g_ref, o_ref):
        o_ref[...] = g_ref[...].reshape(BB, S, D1, D2)

    tc_relayout = pl.pallas_call(
        relayout_body,
        grid=(B // BB,),
        in_specs=[
            pl.BlockSpec((BB * S, D), lambda i: (i, 0)),
        ],
        out_specs=pl.BlockSpec((BB, S, D1, D2), lambda i: (i, 0, 0, 0)),
        out_shape=jax.ShapeDtypeStruct((B, S, D1, D2), weight.dtype),
        compiler_params=pltpu.CompilerParams(
            dimension_semantics=("parallel",),
        ),
    )

    gathered = sc_gather(table, idx)
    return tc_relayout(gathered)
